# baseline (speedup 1.0000x reference)
import functools

import jax
import jax.numpy as jnp
from jax import lax
from jax.experimental import pallas as pl
from jax.experimental.pallas import tpu as pltpu
from jax.experimental.pallas import tpu_sc as plsc

NN = 50000
EE = 800000
CH = 64
HALF = 32
GG = 64
BN_SCALE = 1.0 / (1.0 + 1e-5) ** 0.5

EPAD = 819200
PAD_E = EPAD - EE
IDXW = 128
CHUNK_ROWS = 4
SLOT_ROWS = 2
TILES = 16
ROWS_PER_TILE = EPAD // IDXW // TILES
NSUB = ROWS_PER_TILE // SLOT_ROWS
NP = 50048
ACC_ROWS = NP
ZROWS_PER_TILE = ACC_ROWS // TILES

RB = 3128
NBLK = NP // RB


def _sc_agg_body(h_lo, h_hi, src2d, dst2d, zeros, agg_lo, agg_hi,
                 acc, src_idx, dst_idx, rows, sem):
    c = lax.axis_index("c")
    s = lax.axis_index("s")
    pltpu.sync_copy(zeros.at[pl.ds(s * ZROWS_PER_TILE, ZROWS_PER_TILE)],
                    acc.at[pl.ds(s * ZROWS_PER_TILE, ZROWS_PER_TILE)])
    plsc.subcore_barrier()

    def run(h_ref):
        tile_base = s * ROWS_PER_TILE

        def load_idx(slot, crow):
            pltpu.sync_copy(src2d.at[pl.ds(crow, SLOT_ROWS)],
                            src_idx.at[pl.ds(slot * SLOT_ROWS, SLOT_ROWS)])
            pltpu.sync_copy(dst2d.at[pl.ds(crow, SLOT_ROWS)],
                            dst_idx.at[pl.ds(slot * SLOT_ROWS, SLOT_ROWS)])

        def fire_g(slot):
            for j in range(SLOT_ROWS):
                r = slot * SLOT_ROWS + j
                pltpu.async_copy(h_ref.at[src_idx.at[r]],
                                 rows.at[pl.ds(r * IDXW, IDXW)], sem)

        def wait_g(slot):
            for j in range(SLOT_ROWS):
                r = slot * SLOT_ROWS + j
                pltpu.make_async_copy(h_ref.at[src_idx.at[r]],
                                      rows.at[pl.ds(r * IDXW, IDXW)],
                                      sem).wait()

        def scatter(slot):
            for j in range(SLOT_ROWS):
                r = slot * SLOT_ROWS + j
                pltpu.sync_copy(rows.at[pl.ds(r * IDXW, IDXW)],
                                acc.at[dst_idx.at[r]], add=True)

        load_idx(0, tile_base)
        fire_g(0)

        def section(slot, ci):
            wait_g(slot)

            @pl.when(ci + 1 < NSUB)
            def _():
                load_idx(1 - slot, tile_base + (ci + 1) * SLOT_ROWS)
                fire_g(1 - slot)

            scatter(slot)

        def body(k, carry):
            section(0, 2 * k)
            section(1, 2 * k + 1)
            return carry
        lax.fori_loop(0, NSUB // 2, body, 0)

    @pl.when(c == 0)
    def _():
        run(h_lo)

    @pl.when(c == 1)
    def _():
        run(h_hi)

    plsc.subcore_barrier()

    @pl.when(c == 0)
    def _():
        pltpu.sync_copy(acc.at[pl.ds(s * ZROWS_PER_TILE, ZROWS_PER_TILE)],
                        agg_lo.at[pl.ds(s * ZROWS_PER_TILE, ZROWS_PER_TILE)])

    @pl.when(c == 1)
    def _():
        pltpu.sync_copy(acc.at[pl.ds(s * ZROWS_PER_TILE, ZROWS_PER_TILE)],
                        agg_hi.at[pl.ds(s * ZROWS_PER_TILE, ZROWS_PER_TILE)])


@functools.cache
def _get_sc_agg():
    return functools.partial(
        pl.kernel,
        mesh=plsc.VectorSubcoreMesh(core_axis_name="c", subcore_axis_name="s"),
        compiler_params=pltpu.CompilerParams(use_tc_tiling_on_sc=False),
        out_type=(jax.ShapeDtypeStruct((NP, HALF), jnp.float32),
                  jax.ShapeDtypeStruct((NP, HALF), jnp.float32)),
        scratch_types=[
            pltpu.VMEM_SHARED((ACC_ROWS, HALF), jnp.float32),
            pltpu.VMEM((CHUNK_ROWS, IDXW), jnp.int32),
            pltpu.VMEM((CHUNK_ROWS, IDXW), jnp.int32),
            pltpu.VMEM((CHUNK_ROWS * IDXW, HALF), jnp.float32),
            pltpu.SemaphoreType.DMA,
        ],
    )(_sc_agg_body)


def _proj_body(x_ref, w0_ref, b0_ref, ol_ref, oh_ref):
    xv = x_ref[...]
    w0 = w0_ref[...]
    h = xv[:, 0:1] * w0[0:1, :] + xv[:, 1:2] * w0[1:2, :] + b0_ref[...]
    ol_ref[...] = h[:, :HALF]
    oh_ref[...] = h[:, HALF:]


_proj = pl.pallas_call(
    _proj_body,
    grid=(NBLK,),
    in_specs=[
        pl.BlockSpec((RB, 2), lambda i: (i, 0)),
        pl.BlockSpec((2, CH), lambda i: (0, 0)),
        pl.BlockSpec((1, CH), lambda i: (0, 0)),
    ],
    out_specs=[
        pl.BlockSpec((RB, HALF), lambda i: (i, 0)),
        pl.BlockSpec((RB, HALF), lambda i: (i, 0)),
    ],
    out_shape=[
        jax.ShapeDtypeStruct((NP, HALF), jnp.float32),
        jax.ShapeDtypeStruct((NP, HALF), jnp.float32),
    ],
)


def _mlp_body(hl_ref, hh_ref, al_ref, ah_ref, w1_ref, b1_ref, g_ref, bt_ref,
              w2_ref, b2_ref, ol_ref, oh_ref):
    z = jnp.concatenate([hl_ref[...] + al_ref[...],
                         hh_ref[...] + ah_ref[...]], axis=1)
    z = jnp.dot(z, w1_ref[...], preferred_element_type=jnp.float32)
    z = z + b1_ref[...]
    z = z * (g_ref[...] * BN_SCALE) + bt_ref[...]
    z = jnp.maximum(z, 0.0)
    z = jnp.dot(z, w2_ref[...], preferred_element_type=jnp.float32)
    z = z + b2_ref[...]
    z = jnp.maximum(z, 0.0)
    ol_ref[...] = z[:, :HALF]
    oh_ref[...] = z[:, HALF:]


_mlp = pl.pallas_call(
    _mlp_body,
    grid=(NBLK,),
    in_specs=[
        pl.BlockSpec((RB, HALF), lambda i: (i, 0)),
        pl.BlockSpec((RB, HALF), lambda i: (i, 0)),
        pl.BlockSpec((RB, HALF), lambda i: (i, 0)),
        pl.BlockSpec((RB, HALF), lambda i: (i, 0)),
        pl.BlockSpec((CH, 2 * CH), lambda i: (0, 0)),
        pl.BlockSpec((1, 2 * CH), lambda i: (0, 0)),
        pl.BlockSpec((1, 2 * CH), lambda i: (0, 0)),
        pl.BlockSpec((1, 2 * CH), lambda i: (0, 0)),
        pl.BlockSpec((2 * CH, CH), lambda i: (0, 0)),
        pl.BlockSpec((1, CH), lambda i: (0, 0)),
    ],
    out_specs=[
        pl.BlockSpec((RB, HALF), lambda i: (i, 0)),
        pl.BlockSpec((RB, HALF), lambda i: (i, 0)),
    ],
    out_shape=[
        jax.ShapeDtypeStruct((NP, HALF), jnp.float32),
        jax.ShapeDtypeStruct((NP, HALF), jnp.float32),
    ],
)


def _pool_body(hl_ref, hh_ref, b_ref, pw1_ref, pb1_ref, pw2_ref, pb2_ref,
               out_ref, sums, cnt):
    i = pl.program_id(0)

    @pl.when(i == 0)
    def _():
        sums[...] = jnp.zeros_like(sums)
        cnt[...] = jnp.zeros_like(cnt)

    b = b_ref[...][0]
    iota = lax.broadcasted_iota(jnp.int32, (GG, RB), 0)
    oh_t = (iota == b).astype(jnp.float32)
    h = jnp.concatenate([hl_ref[...], hh_ref[...]], axis=1)
    sums[...] += jnp.dot(oh_t, h, preferred_element_type=jnp.float32)
    cnt[...] += jnp.sum(oh_t, axis=1, keepdims=True)

    @pl.when(i == NBLK - 1)
    def _():
        pooled = sums[...] / jnp.maximum(cnt[...], 1.0)
        y = jnp.dot(pooled, pw1_ref[...], preferred_element_type=jnp.float32)
        y = y + pb1_ref[...]
        y = jnp.where(y > 0, y, jnp.exp(jnp.minimum(y, 0.0)) - 1.0)
        y = jnp.dot(y, pw2_ref[...], preferred_element_type=jnp.float32)
        y = y + pb2_ref[...]
        m = jnp.max(y, axis=1, keepdims=True)
        e = y - m
        lse = jnp.log(jnp.sum(jnp.exp(e), axis=1, keepdims=True))
        out_ref[...] = e - lse


_pool = pl.pallas_call(
    _pool_body,
    grid=(NBLK,),
    in_specs=[
        pl.BlockSpec((RB, HALF), lambda i: (i, 0)),
        pl.BlockSpec((RB, HALF), lambda i: (i, 0)),
        pl.BlockSpec((1, 1, RB), lambda i: (i, 0, 0)),
        pl.BlockSpec((CH, HALF), lambda i: (0, 0)),
        pl.BlockSpec((1, HALF), lambda i: (0, 0)),
        pl.BlockSpec((HALF, 2), lambda i: (0, 0)),
        pl.BlockSpec((1, 2), lambda i: (0, 0)),
    ],
    out_specs=pl.BlockSpec((GG, 2), lambda i: (0, 0)),
    out_shape=jax.ShapeDtypeStruct((GG, 2), jnp.float32),
    scratch_shapes=[
        pltpu.VMEM((GG, CH), jnp.float32),
        pltpu.VMEM((GG, 1), jnp.float32),
    ],
)


def kernel(x, edge_index, batch0, w0, b0,
           l0_w1, l0_b1, l0_gamma, l0_beta, l0_w2, l0_b2,
           l1_w1, l1_b1, l1_gamma, l1_beta, l1_w2, l1_b2,
           l2_w1, l2_b1, l2_gamma, l2_beta, l2_w2, l2_b2,
           p_w1, p_b1, p_w2, p_b2):
    src = edge_index[0]
    dst = edge_index[1]
    src2d = jnp.concatenate(
        [src, jnp.zeros((PAD_E,), jnp.int32)]).reshape(EPAD // IDXW, IDXW)
    dst2d = jnp.concatenate(
        [dst, jnp.full((PAD_E,), NN, jnp.int32)]).reshape(EPAD // IDXW, IDXW)
    zeros = jnp.zeros((ACC_ROWS, HALF), jnp.float32)
    xp = jnp.concatenate([x, jnp.zeros((NP - NN, 2), jnp.float32)])
    bp = jnp.concatenate([batch0, jnp.full((NP - NN,), GG, jnp.int32)])
    b3d = bp.reshape(NBLK, 1, RB)

    hl, hh = _proj(xp, w0, b0.reshape(1, CH))
    layers = [
        (l0_w1, l0_b1, l0_gamma, l0_beta, l0_w2, l0_b2),
        (l1_w1, l1_b1, l1_gamma, l1_beta, l1_w2, l1_b2),
        (l2_w1, l2_b1, l2_gamma, l2_beta, l2_w2, l2_b2),
    ]
    sc_agg = _get_sc_agg()
    for (w1, b1, g, bt, w2, b2) in layers:
        al, ah = sc_agg(hl, hh, src2d, dst2d, zeros)
        hl, hh = _mlp(hl, hh, al, ah, w1, b1.reshape(1, -1),
                      g.reshape(1, -1), bt.reshape(1, -1), w2,
                      b2.reshape(1, -1))
    return _pool(hl, hh, b3d, p_w1, p_b1.reshape(1, -1), p_w2,
                 p_b2.reshape(1, -1))


# speedup vs baseline: 1.1170x; 1.1170x over previous
import functools

import jax
import jax.numpy as jnp
from jax import lax
from jax.experimental import pallas as pl
from jax.experimental.pallas import tpu as pltpu
from jax.experimental.pallas import tpu_sc as plsc

NN = 50000
EE = 800000
CH = 64
HALF = 32
GG = 64
BN_SCALE = 1.0 / (1.0 + 1e-5) ** 0.5

EPAD = 819200
PAD_E = EPAD - EE
IDXW = 128
SLOT_ROWS = 5
CHUNK_ROWS = SLOT_ROWS
TILES = 16
ROWS_PER_TILE = EPAD // IDXW // TILES
NSUB = ROWS_PER_TILE // SLOT_ROWS
NP = 50048
ACC_ROWS = NP
ZROWS_PER_TILE = ACC_ROWS // TILES

RB = 3128
NBLK = NP // RB


def _sc_agg_body(h_lo, h_hi, src2d, dst2d, zeros, agg_lo, agg_hi,
                 acc, src_idx, dst_idx, rows, sem):
    c = lax.axis_index("c")
    s = lax.axis_index("s")
    pltpu.sync_copy(zeros.at[pl.ds(s * ZROWS_PER_TILE, ZROWS_PER_TILE)],
                    acc.at[pl.ds(s * ZROWS_PER_TILE, ZROWS_PER_TILE)])
    plsc.subcore_barrier()

    def run(h_ref):
        tile_base = s * ROWS_PER_TILE

        def load_idx(slot, crow):
            pltpu.sync_copy(src2d.at[pl.ds(crow, SLOT_ROWS)],
                            src_idx.at[pl.ds(slot * 8, SLOT_ROWS)])
            pltpu.sync_copy(dst2d.at[pl.ds(crow, SLOT_ROWS)],
                            dst_idx.at[pl.ds(slot * 8, SLOT_ROWS)])

        def body(ci, carry):
            load_idx(0, tile_base + ci * SLOT_ROWS)
            cps = [pltpu.async_copy(h_ref.at[src_idx.at[j]],
                                    rows.at[pl.ds(j * IDXW, IDXW)], sem)
                   for j in range(SLOT_ROWS)]
            for cp in cps:
                cp.wait()
            for j in range(SLOT_ROWS):
                pltpu.sync_copy(rows.at[pl.ds(j * IDXW, IDXW)],
                                acc.at[dst_idx.at[j]],
                                add=True)
            return carry
        lax.fori_loop(0, NSUB, body, 0)

    @pl.when(c == 0)
    def _():
        run(h_lo)

    @pl.when(c == 1)
    def _():
        run(h_hi)

    plsc.subcore_barrier()

    @pl.when(c == 0)
    def _():
        pltpu.sync_copy(acc.at[pl.ds(s * ZROWS_PER_TILE, ZROWS_PER_TILE)],
                        agg_lo.at[pl.ds(s * ZROWS_PER_TILE, ZROWS_PER_TILE)])

    @pl.when(c == 1)
    def _():
        pltpu.sync_copy(acc.at[pl.ds(s * ZROWS_PER_TILE, ZROWS_PER_TILE)],
                        agg_hi.at[pl.ds(s * ZROWS_PER_TILE, ZROWS_PER_TILE)])


@functools.cache
def _get_sc_agg():
    return functools.partial(
        pl.kernel,
        mesh=plsc.VectorSubcoreMesh(core_axis_name="c", subcore_axis_name="s"),
        compiler_params=pltpu.CompilerParams(use_tc_tiling_on_sc=False),
        out_type=(jax.ShapeDtypeStruct((NP, HALF), jnp.float32),
                  jax.ShapeDtypeStruct((NP, HALF), jnp.float32)),
        scratch_types=[
            pltpu.VMEM_SHARED((ACC_ROWS, HALF), jnp.float32),
            pltpu.VMEM((16, IDXW), jnp.int32),
            pltpu.VMEM((16, IDXW), jnp.int32),
            pltpu.VMEM((SLOT_ROWS * IDXW, HALF), jnp.float32),
            pltpu.SemaphoreType.DMA,
        ],
    )(_sc_agg_body)


def _proj_body(x_ref, w0_ref, b0_ref, ol_ref, oh_ref):
    xv = x_ref[...]
    w0 = w0_ref[...]
    h = xv[:, 0:1] * w0[0:1, :] + xv[:, 1:2] * w0[1:2, :] + b0_ref[...]
    ol_ref[...] = h[:, :HALF]
    oh_ref[...] = h[:, HALF:]


_proj = pl.pallas_call(
    _proj_body,
    grid=(NBLK,),
    in_specs=[
        pl.BlockSpec((RB, 2), lambda i: (i, 0)),
        pl.BlockSpec((2, CH), lambda i: (0, 0)),
        pl.BlockSpec((1, CH), lambda i: (0, 0)),
    ],
    out_specs=[
        pl.BlockSpec((RB, HALF), lambda i: (i, 0)),
        pl.BlockSpec((RB, HALF), lambda i: (i, 0)),
    ],
    out_shape=[
        jax.ShapeDtypeStruct((NP, HALF), jnp.float32),
        jax.ShapeDtypeStruct((NP, HALF), jnp.float32),
    ],
)


def _mlp_body(hl_ref, hh_ref, al_ref, ah_ref, w1_ref, b1_ref, g_ref, bt_ref,
              w2_ref, b2_ref, ol_ref, oh_ref):
    z = jnp.concatenate([hl_ref[...] + al_ref[...],
                         hh_ref[...] + ah_ref[...]], axis=1)
    z = jnp.dot(z, w1_ref[...], preferred_element_type=jnp.float32)
    z = z + b1_ref[...]
    z = z * (g_ref[...] * BN_SCALE) + bt_ref[...]
    z = jnp.maximum(z, 0.0)
    z = jnp.dot(z, w2_ref[...], preferred_element_type=jnp.float32)
    z = z + b2_ref[...]
    z = jnp.maximum(z, 0.0)
    ol_ref[...] = z[:, :HALF]
    oh_ref[...] = z[:, HALF:]


_mlp = pl.pallas_call(
    _mlp_body,
    grid=(NBLK,),
    in_specs=[
        pl.BlockSpec((RB, HALF), lambda i: (i, 0)),
        pl.BlockSpec((RB, HALF), lambda i: (i, 0)),
        pl.BlockSpec((RB, HALF), lambda i: (i, 0)),
        pl.BlockSpec((RB, HALF), lambda i: (i, 0)),
        pl.BlockSpec((CH, 2 * CH), lambda i: (0, 0)),
        pl.BlockSpec((1, 2 * CH), lambda i: (0, 0)),
        pl.BlockSpec((1, 2 * CH), lambda i: (0, 0)),
        pl.BlockSpec((1, 2 * CH), lambda i: (0, 0)),
        pl.BlockSpec((2 * CH, CH), lambda i: (0, 0)),
        pl.BlockSpec((1, CH), lambda i: (0, 0)),
    ],
    out_specs=[
        pl.BlockSpec((RB, HALF), lambda i: (i, 0)),
        pl.BlockSpec((RB, HALF), lambda i: (i, 0)),
    ],
    out_shape=[
        jax.ShapeDtypeStruct((NP, HALF), jnp.float32),
        jax.ShapeDtypeStruct((NP, HALF), jnp.float32),
    ],
)


def _pool_body(hl_ref, hh_ref, b_ref, pw1_ref, pb1_ref, pw2_ref, pb2_ref,
               out_ref, sums, cnt):
    i = pl.program_id(0)

    @pl.when(i == 0)
    def _():
        sums[...] = jnp.zeros_like(sums)
        cnt[...] = jnp.zeros_like(cnt)

    b = b_ref[...][0]
    iota = lax.broadcasted_iota(jnp.int32, (GG, RB), 0)
    oh_t = (iota == b).astype(jnp.float32)
    h = jnp.concatenate([hl_ref[...], hh_ref[...]], axis=1)
    sums[...] += jnp.dot(oh_t, h, preferred_element_type=jnp.float32)
    cnt[...] += jnp.sum(oh_t, axis=1, keepdims=True)

    @pl.when(i == NBLK - 1)
    def _():
        pooled = sums[...] / jnp.maximum(cnt[...], 1.0)
        y = jnp.dot(pooled, pw1_ref[...], preferred_element_type=jnp.float32)
        y = y + pb1_ref[...]
        y = jnp.where(y > 0, y, jnp.exp(jnp.minimum(y, 0.0)) - 1.0)
        y = jnp.dot(y, pw2_ref[...], preferred_element_type=jnp.float32)
        y = y + pb2_ref[...]
        m = jnp.max(y, axis=1, keepdims=True)
        e = y - m
        lse = jnp.log(jnp.sum(jnp.exp(e), axis=1, keepdims=True))
        out_ref[...] = e - lse


_pool = pl.pallas_call(
    _pool_body,
    grid=(NBLK,),
    in_specs=[
        pl.BlockSpec((RB, HALF), lambda i: (i, 0)),
        pl.BlockSpec((RB, HALF), lambda i: (i, 0)),
        pl.BlockSpec((1, 1, RB), lambda i: (i, 0, 0)),
        pl.BlockSpec((CH, HALF), lambda i: (0, 0)),
        pl.BlockSpec((1, HALF), lambda i: (0, 0)),
        pl.BlockSpec((HALF, 2), lambda i: (0, 0)),
        pl.BlockSpec((1, 2), lambda i: (0, 0)),
    ],
    out_specs=pl.BlockSpec((GG, 2), lambda i: (0, 0)),
    out_shape=jax.ShapeDtypeStruct((GG, 2), jnp.float32),
    scratch_shapes=[
        pltpu.VMEM((GG, CH), jnp.float32),
        pltpu.VMEM((GG, 1), jnp.float32),
    ],
)


def kernel(x, edge_index, batch0, w0, b0,
           l0_w1, l0_b1, l0_gamma, l0_beta, l0_w2, l0_b2,
           l1_w1, l1_b1, l1_gamma, l1_beta, l1_w2, l1_b2,
           l2_w1, l2_b1, l2_gamma, l2_beta, l2_w2, l2_b2,
           p_w1, p_b1, p_w2, p_b2):
    src = edge_index[0]
    dst = edge_index[1]
    src2d = jnp.concatenate(
        [src, jnp.zeros((PAD_E,), jnp.int32)]).reshape(EPAD // IDXW, IDXW)
    dst2d = jnp.concatenate(
        [dst, jnp.full((PAD_E,), NN, jnp.int32)]).reshape(EPAD // IDXW, IDXW)
    zeros = jnp.zeros((ACC_ROWS, HALF), jnp.float32)
    xp = jnp.concatenate([x, jnp.zeros((NP - NN, 2), jnp.float32)])
    bp = jnp.concatenate([batch0, jnp.full((NP - NN,), GG, jnp.int32)])
    b3d = bp.reshape(NBLK, 1, RB)

    hl, hh = _proj(xp, w0, b0.reshape(1, CH))
    layers = [
        (l0_w1, l0_b1, l0_gamma, l0_beta, l0_w2, l0_b2),
        (l1_w1, l1_b1, l1_gamma, l1_beta, l1_w2, l1_b2),
        (l2_w1, l2_b1, l2_gamma, l2_beta, l2_w2, l2_b2),
    ]
    sc_agg = _get_sc_agg()
    for (w1, b1, g, bt, w2, b2) in layers:
        al, ah = sc_agg(hl, hh, src2d, dst2d, zeros)
        hl, hh = _mlp(hl, hh, al, ah, w1, b1.reshape(1, -1),
                      g.reshape(1, -1), bt.reshape(1, -1), w2,
                      b2.reshape(1, -1))
    return _pool(hl, hh, b3d, p_w1, p_b1.reshape(1, -1), p_w2,
                 p_b2.reshape(1, -1))
